# speedup vs baseline: 1.4566x; 1.4566x over previous
import functools
import math

import jax
import jax.numpy as jnp
from jax import lax
from jax.experimental import pallas as pl
from jax.experimental.pallas import tpu as pltpu
from jax.experimental.pallas import tpu_sc as plsc

D_MODEL = 1024
SCALE = math.sqrt(D_MODEL)

_INFO = plsc.get_sparse_core_info()
NC, NS, L = _INFO.num_cores, _INFO.num_subcores, _INFO.num_lanes
NW = NC * NS

CH = 32
NBUF = 2


def _emb_body(b_per_w, n_chunk, x_hbm, table_hbm, out_hbm,
              idx_v, rows_v, sem0, sem1):
    sems = (sem0, sem1)
    wid = lax.axis_index("s") * NC + lax.axis_index("c")
    base = wid * b_per_w

    pltpu.sync_copy(x_hbm.at[pl.ds(base, b_per_w)], idx_v)

    def start_gather(c, b):
        pltpu.make_async_copy(
            table_hbm.at[idx_v.at[pl.ds(c * CH, CH)]],
            rows_v.at[b], sems[b]).start()

    def wait_gather(c, b):
        pltpu.make_async_copy(
            table_hbm.at[idx_v.at[pl.ds(c * CH, CH)]],
            rows_v.at[b], sems[b]).wait()

    for b in range(NBUF):
        start_gather(b, b)

    def group(g, _):
        for b in range(NBUF):
            c = g * NBUF + b
            wait_gather(c, b)

            def scale_row(r, _):
                for k in range(D_MODEL // L):
                    rows_v[b, r, pl.ds(k * L, L)] = (
                        rows_v[b, r, pl.ds(k * L, L)] * SCALE)
                return 0

            lax.fori_loop(0, CH, scale_row, 0, unroll=False)
            pltpu.sync_copy(rows_v.at[b], out_hbm.at[pl.ds(base + c * CH, CH)])

            @pl.when(c + NBUF < n_chunk)
            def _():
                start_gather(c + NBUF, b)
        return 0

    lax.fori_loop(0, n_chunk // NBUF, group, 0, unroll=False)


def kernel(x, table):
    orig_shape = x.shape
    xf = x.reshape(-1).astype(jnp.int32)
    b_total = xf.shape[0]
    b_per_w = b_total // NW
    n_chunk = b_per_w // CH

    mesh = plsc.VectorSubcoreMesh(core_axis_name="c", subcore_axis_name="s")
    k = pl.kernel(
        functools.partial(_emb_body, b_per_w, n_chunk),
        mesh=mesh,
        out_type=jax.ShapeDtypeStruct((b_total, D_MODEL), jnp.float32),
        scratch_types=[
            pltpu.VMEM((b_per_w,), jnp.int32),
            pltpu.VMEM((NBUF, CH, D_MODEL), jnp.float32),
            pltpu.SemaphoreType.DMA,
            pltpu.SemaphoreType.DMA,
        ],
    )
    out = k(xf, table)
    return out.reshape(*orig_shape, D_MODEL)
